# baseline (speedup 1.0000x reference)
import functools

import jax
import jax.numpy as jnp
from jax import lax
from jax.experimental import pallas as pl
from jax.experimental.pallas import tpu as pltpu
from jax.experimental.pallas import tpu_sc as plsc

N_NODE = 10000
NPAD = 10240
D = 128
E_EDGE = 320000
E_LBL = 100000

NSC = 2
NTILE = 16
CH = 80
K_CONV = 256
G_CONV = 8
NB = 4
EPT = NTILE * K_CONV * CH
STRIPE = NPAD // NTILE

VLBL = 102400
K_LBL = VLBL // (NSC * NTILE * CH)


def _zero_fill_2d(ref, rows):
    def row(r, c):
        for i in range(D // 16):
            ref[r, pl.ds(i * 16, 16)] = jnp.zeros((16,), jnp.float32)
        return c
    lax.fori_loop(0, rows, row, 0)


def _conv_body(with_counts, x_hbm, src_hbm, dst_hbm, *rest):
    if with_counts:
        (agg_out, cnt_out, srcv, dstv, bufs, ones_v, cz_v,
         agg_sh, cnt_sh, sem_g, sem_s, sem_is, sem_id, sem_c) = rest
    else:
        (agg_out, srcv, dstv, bufs, agg_sh,
         sem_g, sem_s, sem_is, sem_id) = rest
    cid = lax.axis_index("c")
    sid = lax.axis_index("s")
    NG = K_CONV // G_CONV

    _zero_fill_2d(bufs[0], CH)
    for b in range(STRIPE // CH):
        pltpu.sync_copy(bufs[0], agg_sh.at[pl.ds(sid * STRIPE + b * CH, CH)])
    if with_counts:
        for i in range(CH // 16):
            ones_v[pl.ds(i * 16, 16)] = jnp.ones((16,), jnp.float32)
        def zc(i, c):
            cz_v[pl.ds(i * 16, 16)] = jnp.zeros((16,), jnp.float32)
            return c
        lax.fori_loop(0, STRIPE // 16, zc, 0)
        pltpu.sync_copy(cz_v, cnt_sh.at[pl.ds(sid * STRIPE, STRIPE)])
    plsc.subcore_barrier()

    def count_drain(n):
        def one(i, c):
            pltpu.make_async_copy(ones_v, cnt_sh.at[dstv.at[0, 0]],
                                  sem_c).wait()
            return c
        lax.fori_loop(0, n, one, 0)

    pltpu.sync_copy(src_hbm.at[cid, sid, pl.ds(0, G_CONV)], srcv.at[0])
    pltpu.sync_copy(dst_hbm.at[cid, sid, pl.ds(0, G_CONV)], dstv.at[0])
    for b in range(2):
        pltpu.async_copy(x_hbm.at[srcv.at[0, b]], bufs[b], sem_g[b])

    def quad(j4, c):
        for slot in range(NB):
            j = NB * j4 + slot
            buf = bufs[slot]
            g = j // G_CONV
            r = j - g * G_CONV
            p = lax.rem(g, 2)

            if with_counts:
                @pl.when(jnp.logical_and(r == 0, g >= 1))
                def _():
                    count_drain(G_CONV)

            @pl.when(jnp.logical_and(r == 0, g < NG - 1))
            def _():
                pltpu.async_copy(
                    src_hbm.at[cid, sid, pl.ds((g + 1) * G_CONV, G_CONV)],
                    srcv.at[1 - p], sem_is)
                pltpu.async_copy(
                    dst_hbm.at[cid, sid, pl.ds((g + 1) * G_CONV, G_CONV)],
                    dstv.at[1 - p], sem_id)

            pltpu.make_async_copy(x_hbm.at[srcv.at[p, r]], buf, sem_g[slot]).wait()
            pltpu.async_copy(buf, agg_sh.at[dstv.at[p, r]], sem_s[slot],
                             add=True)
            if with_counts:
                pltpu.async_copy(ones_v, cnt_sh.at[dstv.at[p, r]], sem_c,
                                 add=True)

            @pl.when(jnp.logical_and(r == G_CONV - 2, g < NG - 1))
            def _():
                pltpu.make_async_copy(
                    src_hbm.at[cid, sid, pl.ds(0, G_CONV)],
                    srcv.at[1 - p], sem_is).wait()

            @pl.when(jnp.logical_and(r == G_CONV - 1, g < NG - 1))
            def _():
                pltpu.make_async_copy(
                    dst_hbm.at[cid, sid, pl.ds(0, G_CONV)],
                    dstv.at[1 - p], sem_id).wait()

            nslot = (slot + 2) % NB
            @pl.when(j >= 2)
            def _():
                pltpu.make_async_copy(bufs[nslot], agg_sh.at[dstv.at[0, 0]],
                                      sem_s[nslot]).wait()

            jn = j + 2
            gn = jn // G_CONV
            rn = jn - gn * G_CONV
            pn = lax.rem(gn, 2)

            @pl.when(jn < K_CONV)
            def _():
                pltpu.async_copy(x_hbm.at[srcv.at[pn, rn]], bufs[nslot],
                                 sem_g[nslot])
        return c
    lax.fori_loop(0, K_CONV // NB, quad, 0)

    for j in (K_CONV - 2, K_CONV - 1):
        pltpu.make_async_copy(bufs[j % NB], agg_sh.at[dstv.at[0, 0]],
                              sem_s[j % NB]).wait()
    if with_counts:
        count_drain(G_CONV)
    plsc.subcore_barrier()

    base = (1 - cid) * NPAD + sid * STRIPE
    pltpu.sync_copy(agg_sh.at[pl.ds(sid * STRIPE, STRIPE)],
                    agg_out.at[pl.ds(base, STRIPE)])
    if with_counts:
        pltpu.sync_copy(cnt_sh.at[pl.ds(sid * STRIPE, STRIPE)],
                        cnt_out.at[pl.ds(base, STRIPE)])


def _make_conv(with_counts):
    mesh = plsc.VectorSubcoreMesh(core_axis_name="c", subcore_axis_name="s")
    out_type = [jax.ShapeDtypeStruct((2 * NPAD, D), jnp.float32)]
    scratch = [
        pltpu.VMEM((2, G_CONV, CH), jnp.int32),
        pltpu.VMEM((2, G_CONV, CH), jnp.int32),
        tuple(pltpu.VMEM((CH, D), jnp.float32) for _ in range(NB)),
    ]
    if with_counts:
        out_type.append(jax.ShapeDtypeStruct((2 * NPAD,), jnp.float32))
        scratch += [
            pltpu.VMEM((CH,), jnp.float32),
            pltpu.VMEM((STRIPE,), jnp.float32),
        ]
    scratch.append(pltpu.VMEM_SHARED((NPAD, D), jnp.float32))
    if with_counts:
        scratch.append(pltpu.VMEM_SHARED((NPAD,), jnp.float32))
    scratch += [
        tuple(pltpu.SemaphoreType.DMA for _ in range(NB)),
        tuple(pltpu.SemaphoreType.DMA for _ in range(NB)),
        pltpu.SemaphoreType.DMA,
        pltpu.SemaphoreType.DMA,
    ]
    if with_counts:
        scratch.append(pltpu.SemaphoreType.DMA)
    return pl.kernel(
        functools.partial(_conv_body, with_counts),
        out_type=tuple(out_type),
        mesh=mesh,
        scratch_types=tuple(scratch),
    )


def _decoder_body(u_hbm, row_hbm, col_hbm, v_out,
                  row_v, col_v, bufs, sem_r, sem_a, sem_w):
    cid = lax.axis_index("c")
    sid = lax.axis_index("s")
    w = cid * NTILE + sid
    base = w * (K_LBL * CH)
    pltpu.sync_copy(row_hbm.at[cid, sid], row_v)
    pltpu.sync_copy(col_hbm.at[cid, sid], col_v)
    for b in range(2):
        pltpu.async_copy(u_hbm.at[row_v.at[b]], bufs[b], sem_r[b])

    def quad(j4, c):
        for slot in range(NB):
            j = NB * j4 + slot

            @pl.when(j < K_LBL)
            def _():
                pltpu.make_async_copy(u_hbm.at[row_v.at[j]], bufs[slot],
                                      sem_r[slot]).wait()
                pltpu.async_copy(u_hbm.at[col_v.at[j]], bufs[slot],
                                 sem_a[slot], add=True)

            ja = j - 1
            aslot = (slot - 1) % NB

            @pl.when(jnp.logical_and(ja >= 0, ja < K_LBL))
            def _():
                pltpu.make_async_copy(u_hbm.at[col_v.at[ja]], bufs[aslot],
                                      sem_a[aslot]).wait()
                pltpu.async_copy(bufs[aslot],
                                 v_out.at[pl.ds(base + ja * CH, CH)],
                                 sem_w[aslot])

            jw = j - 2
            wslot = (slot - 2) % NB

            @pl.when(jnp.logical_and(jw >= 0, jw < K_LBL))
            def _():
                pltpu.make_async_copy(bufs[wslot],
                                      v_out.at[pl.ds(base, CH)],
                                      sem_w[wslot]).wait()

            jn = j + 2

            @pl.when(jn < K_LBL)
            def _():
                pltpu.async_copy(u_hbm.at[row_v.at[jn]], bufs[wslot],
                                 sem_r[wslot])
        return c
    lax.fori_loop(0, (K_LBL + 2 + NB - 1) // NB, quad, 0)


_decoder_sc = pl.kernel(
    _decoder_body,
    out_type=jax.ShapeDtypeStruct((VLBL, D), jnp.float32),
    mesh=plsc.VectorSubcoreMesh(core_axis_name="c", subcore_axis_name="s"),
    scratch_types=(
        pltpu.VMEM((K_LBL, CH), jnp.int32),
        pltpu.VMEM((K_LBL, CH), jnp.int32),
        tuple(pltpu.VMEM((CH, D), jnp.float32) for _ in range(NB)),
        tuple(pltpu.SemaphoreType.DMA for _ in range(NB)),
        tuple(pltpu.SemaphoreType.DMA for _ in range(NB)),
        tuple(pltpu.SemaphoreType.DMA for _ in range(NB)),
    ),
)


_BLK = 1024


def _layer1_tc_body(agg_ref, cnt_ref, x_ref, wl_ref, bl_ref, wr_ref, o_ref):
    inv = 1.0 / jnp.maximum(cnt_ref[...], 1.0)
    mean = agg_ref[...] * inv
    h = (jnp.dot(mean, wl_ref[0], preferred_element_type=jnp.float32)
         + bl_ref[0]
         + jnp.dot(x_ref[...], wr_ref[0], preferred_element_type=jnp.float32))
    o_ref[...] = jnp.maximum(h, 0.0)


def _layer2_tc_body(agg_ref, cnt_ref, x_ref, wl_ref, bl_ref, wr_ref,
                    w1_ref, b1_ref, o_ref):
    inv = 1.0 / jnp.maximum(cnt_ref[...], 1.0)
    mean = agg_ref[...] * inv
    z = (jnp.dot(mean, wl_ref[0], preferred_element_type=jnp.float32)
         + bl_ref[0]
         + jnp.dot(x_ref[...], wr_ref[0], preferred_element_type=jnp.float32))
    o_ref[...] = jnp.dot(z, w1_ref[0], preferred_element_type=jnp.float32) + b1_ref[0]


def _row_blk(b):
    return (b, 0)


def _w_blk(b):
    return (b // (NPAD // _BLK), 0, 0)


_N_ROWS = 2 * NPAD
_node_specs = [
    pl.BlockSpec((_BLK, D), _row_blk),
    pl.BlockSpec((_BLK, 1), _row_blk),
    pl.BlockSpec((_BLK, D), _row_blk),
    pl.BlockSpec((1, D, D), _w_blk),
    pl.BlockSpec((1, 1, D), _w_blk),
    pl.BlockSpec((1, D, D), _w_blk),
]

_layer1_tc = pl.pallas_call(
    _layer1_tc_body,
    grid=(_N_ROWS // _BLK,),
    in_specs=_node_specs,
    out_specs=pl.BlockSpec((_BLK, D), _row_blk),
    out_shape=jax.ShapeDtypeStruct((_N_ROWS, D), jnp.float32),
)

_layer2_tc = pl.pallas_call(
    _layer2_tc_body,
    grid=(_N_ROWS // _BLK,),
    in_specs=_node_specs + [
        pl.BlockSpec((1, D, D), _w_blk),
        pl.BlockSpec((1, 1, D), _w_blk),
    ],
    out_specs=pl.BlockSpec((_BLK, D), _row_blk),
    out_shape=jax.ShapeDtypeStruct((_N_ROWS, D), jnp.float32),
)


def _final_tc_body(v_ref, w2_ref, b2_ref, o_ref):
    o_ref[...] = (jnp.sum(jnp.maximum(v_ref[...], 0.0) * w2_ref[...],
                          axis=1, keepdims=True) + b2_ref[0, 0])


_FBLK = 2048
_final_tc = pl.pallas_call(
    _final_tc_body,
    grid=(VLBL // _FBLK,),
    in_specs=[
        pl.BlockSpec((_FBLK, D), _row_blk),
        pl.BlockSpec((1, D), lambda b: (0, 0)),
        pl.BlockSpec((1, 1), lambda b: (0, 0)),
    ],
    out_specs=pl.BlockSpec((_FBLK, 1), _row_blk),
    out_shape=jax.ShapeDtypeStruct((VLBL, 1), jnp.float32),
)


def _prep_edges(src, dst, src_off):
    pad = EPT - E_EDGE
    s = jnp.concatenate([src.astype(jnp.int32) + src_off,
                         jnp.full((pad,), src_off, jnp.int32)])
    d = jnp.concatenate([dst.astype(jnp.int32),
                         jnp.full((pad,), N_NODE, jnp.int32)])
    return s.reshape(NTILE, K_CONV, CH), d.reshape(NTILE, K_CONV, CH)


def kernel(x_drug, x_disease, edge_index_dd, edge_index_dr, edge_label_index,
           Wl1_dd, bl1_dd, Wr1_dd, Wl1_dr, bl1_dr, Wr1_dr,
           Wl2_dd, bl2_dd, Wr2_dd, Wl2_dr, bl2_dr, Wr2_dr,
           W1, b1, W2, b2):
    f32 = jnp.float32
    pad_n = NPAD - N_NODE
    x_cat = jnp.concatenate([
        jnp.pad(x_drug, ((0, pad_n), (0, 0))),
        jnp.pad(x_disease, ((0, pad_n), (0, 0))),
    ]).astype(f32)

    s_dd, d_dd = _prep_edges(edge_index_dd[0], edge_index_dd[1], 0)
    s_dr, d_dr = _prep_edges(edge_index_dr[0], edge_index_dr[1], NPAD)
    src_a = jnp.stack([s_dd, s_dr])
    dst_a = jnp.stack([d_dd, d_dr])

    agg1, cnt = _make_conv(True)(x_cat, src_a, dst_a)
    cnt2d = cnt.reshape(-1, 1)

    wl1 = jnp.stack([Wl1_dr, Wl1_dd])
    bl1 = jnp.stack([bl1_dr, bl1_dd]).reshape(2, 1, D)
    wr1 = jnp.stack([Wr1_dr, Wr1_dd])
    h_cat = _layer1_tc(agg1, cnt2d, x_cat, wl1, bl1, wr1)

    (agg2,) = _make_conv(False)(h_cat, src_a, dst_a)
    wl2 = jnp.stack([Wl2_dr, Wl2_dd])
    bl2 = jnp.stack([bl2_dr, bl2_dd]).reshape(2, 1, D)
    wr2 = jnp.stack([Wr2_dr, Wr2_dd])
    w1s = jnp.stack([W1[:D], W1[D:]])
    b1s = jnp.stack([b1, jnp.zeros((D,), f32)]).reshape(2, 1, D)
    u_cat = _layer2_tc(agg2, cnt2d, h_cat, wl2, bl2, wr2, w1s, b1s)

    lpad = VLBL - E_LBL
    row = jnp.concatenate([edge_label_index[0].astype(jnp.int32),
                           jnp.zeros((lpad,), jnp.int32)])
    col = jnp.concatenate([edge_label_index[1].astype(jnp.int32) + NPAD,
                           jnp.full((lpad,), NPAD, jnp.int32)])
    row_a = row.reshape(NSC, NTILE, K_LBL, CH)
    col_a = col.reshape(NSC, NTILE, K_LBL, CH)
    v = _decoder_sc(u_cat, row_a, col_a)

    out = _final_tc(v, W2.reshape(1, D), b2.reshape(1, 1))
    return out[:E_LBL, 0]


# speedup vs baseline: 1.1517x; 1.1517x over previous
import functools

import jax
import jax.numpy as jnp
from jax import lax
from jax.experimental import pallas as pl
from jax.experimental.pallas import tpu as pltpu
from jax.experimental.pallas import tpu_sc as plsc

N_NODE = 10000
NPAD = 10240
D = 128
E_EDGE = 320000
E_LBL = 100000

NSC = 2
NTILE = 16
CH = 128
K_CONV = 160
G_CONV = 16
NB = 4
EPT = NTILE * K_CONV * CH
STRIPE = NPAD // NTILE

VLBL = 102400
K_LBL = VLBL // (NSC * NTILE * CH)


def _zero_fill_2d(ref, rows):
    def row(r, c):
        for i in range(D // 16):
            ref[r, pl.ds(i * 16, 16)] = jnp.zeros((16,), jnp.float32)
        return c
    lax.fori_loop(0, rows, row, 0)


def _conv_body(with_counts, x_hbm, src_hbm, dst_hbm, *rest):
    if with_counts:
        (agg_out, cnt_out, srcv, dstv, bufs, ones_v, cz_v,
         agg_sh, cnt_sh, sem_g, sem_is, sem_id, sem_c) = rest
    else:
        (agg_out, srcv, dstv, bufs, agg_sh,
         sem_g, sem_is, sem_id) = rest
    cid = lax.axis_index("c")
    sid = lax.axis_index("s")
    NG = K_CONV // G_CONV

    _zero_fill_2d(bufs[0], CH)
    for b in range(STRIPE // CH):
        pltpu.sync_copy(bufs[0], agg_sh.at[pl.ds(sid * STRIPE + b * CH, CH)])
    if with_counts:
        for i in range(CH // 16):
            ones_v[pl.ds(i * 16, 16)] = jnp.ones((16,), jnp.float32)
        def zc(i, c):
            cz_v[pl.ds(i * 16, 16)] = jnp.zeros((16,), jnp.float32)
            return c
        lax.fori_loop(0, STRIPE // 16, zc, 0)
        pltpu.sync_copy(cz_v, cnt_sh.at[pl.ds(sid * STRIPE, STRIPE)])
    plsc.subcore_barrier()

    def count_drain(n):
        def one(i, c):
            pltpu.make_async_copy(ones_v, cnt_sh.at[dstv.at[0, 0]],
                                  sem_c).wait()
            return c
        lax.fori_loop(0, n, one, 0)

    pltpu.sync_copy(src_hbm.at[cid, sid, pl.ds(0, G_CONV)], srcv.at[0])
    pltpu.sync_copy(dst_hbm.at[cid, sid, pl.ds(0, G_CONV)], dstv.at[0])
    for b in range(2):
        pltpu.async_copy(x_hbm.at[srcv.at[0, b]], bufs[b], sem_g[b])

    def pair(j2, c):
        for slot in range(2):
            j = 2 * j2 + slot
            buf = bufs[slot]
            g = j // G_CONV
            r = j - g * G_CONV
            p = lax.rem(g, 2)

            if with_counts:
                @pl.when(jnp.logical_and(r == 0, g >= 1))
                def _():
                    count_drain(G_CONV)

            @pl.when(jnp.logical_and(r == 0, g < NG - 1))
            def _():
                pltpu.async_copy(
                    src_hbm.at[cid, sid, pl.ds((g + 1) * G_CONV, G_CONV)],
                    srcv.at[1 - p], sem_is)
                pltpu.async_copy(
                    dst_hbm.at[cid, sid, pl.ds((g + 1) * G_CONV, G_CONV)],
                    dstv.at[1 - p], sem_id)

            pltpu.make_async_copy(x_hbm.at[srcv.at[p, r]], buf,
                                  sem_g[slot]).wait()
            pltpu.sync_copy(buf, agg_sh.at[dstv.at[p, r]], add=True)
            if with_counts:
                pltpu.async_copy(ones_v, cnt_sh.at[dstv.at[p, r]], sem_c,
                                 add=True)

            @pl.when(jnp.logical_and(r == G_CONV - 2, g < NG - 1))
            def _():
                pltpu.make_async_copy(
                    src_hbm.at[cid, sid, pl.ds(0, G_CONV)],
                    srcv.at[1 - p], sem_is).wait()

            @pl.when(jnp.logical_and(r == G_CONV - 1, g < NG - 1))
            def _():
                pltpu.make_async_copy(
                    dst_hbm.at[cid, sid, pl.ds(0, G_CONV)],
                    dstv.at[1 - p], sem_id).wait()

            jn = j + 2
            gn = jn // G_CONV
            rn = jn - gn * G_CONV
            pn = lax.rem(gn, 2)

            @pl.when(jn < K_CONV)
            def _():
                pltpu.async_copy(x_hbm.at[srcv.at[pn, rn]], buf, sem_g[slot])
        return c
    lax.fori_loop(0, K_CONV // 2, pair, 0)

    if with_counts:
        count_drain(G_CONV)
    plsc.subcore_barrier()

    base = (1 - cid) * NPAD + sid * STRIPE
    pltpu.sync_copy(agg_sh.at[pl.ds(sid * STRIPE, STRIPE)],
                    agg_out.at[pl.ds(base, STRIPE)])
    if with_counts:
        pltpu.sync_copy(cnt_sh.at[pl.ds(sid * STRIPE, STRIPE)],
                        cnt_out.at[pl.ds(base, STRIPE)])


def _make_conv(with_counts):
    mesh = plsc.VectorSubcoreMesh(core_axis_name="c", subcore_axis_name="s")
    out_type = [jax.ShapeDtypeStruct((2 * NPAD, D), jnp.float32)]
    scratch = [
        pltpu.VMEM((2, G_CONV, CH), jnp.int32),
        pltpu.VMEM((2, G_CONV, CH), jnp.int32),
        tuple(pltpu.VMEM((CH, D), jnp.float32) for _ in range(2)),
    ]
    if with_counts:
        out_type.append(jax.ShapeDtypeStruct((2 * NPAD,), jnp.float32))
        scratch += [
            pltpu.VMEM((CH,), jnp.float32),
            pltpu.VMEM((STRIPE,), jnp.float32),
        ]
    scratch.append(pltpu.VMEM_SHARED((NPAD, D), jnp.float32))
    if with_counts:
        scratch.append(pltpu.VMEM_SHARED((NPAD,), jnp.float32))
    scratch += [
        tuple(pltpu.SemaphoreType.DMA for _ in range(2)),
        pltpu.SemaphoreType.DMA,
        pltpu.SemaphoreType.DMA,
    ]
    if with_counts:
        scratch.append(pltpu.SemaphoreType.DMA)
    return pl.kernel(
        functools.partial(_conv_body, with_counts),
        out_type=tuple(out_type),
        mesh=mesh,
        scratch_types=tuple(scratch),
    )


def _decoder_body(u_hbm, row_hbm, col_hbm, v_out,
                  row_v, col_v, bufs, sem_r, sem_a, sem_w):
    cid = lax.axis_index("c")
    sid = lax.axis_index("s")
    w = cid * NTILE + sid
    base = w * (K_LBL * CH)
    pltpu.sync_copy(row_hbm.at[cid, sid], row_v)
    pltpu.sync_copy(col_hbm.at[cid, sid], col_v)
    for b in range(2):
        pltpu.async_copy(u_hbm.at[row_v.at[b]], bufs[b], sem_r[b])

    def quad(j4, c):
        for slot in range(NB):
            j = NB * j4 + slot

            @pl.when(j < K_LBL)
            def _():
                pltpu.make_async_copy(u_hbm.at[row_v.at[j]], bufs[slot],
                                      sem_r[slot]).wait()
                pltpu.async_copy(u_hbm.at[col_v.at[j]], bufs[slot],
                                 sem_a[slot], add=True)

            ja = j - 1
            aslot = (slot - 1) % NB

            @pl.when(jnp.logical_and(ja >= 0, ja < K_LBL))
            def _():
                pltpu.make_async_copy(u_hbm.at[col_v.at[ja]], bufs[aslot],
                                      sem_a[aslot]).wait()
                pltpu.async_copy(bufs[aslot],
                                 v_out.at[pl.ds(base + ja * CH, CH)],
                                 sem_w[aslot])

            jw = j - 2
            wslot = (slot - 2) % NB

            @pl.when(jnp.logical_and(jw >= 0, jw < K_LBL))
            def _():
                pltpu.make_async_copy(bufs[wslot],
                                      v_out.at[pl.ds(base, CH)],
                                      sem_w[wslot]).wait()

            jn = j + 2

            @pl.when(jn < K_LBL)
            def _():
                pltpu.async_copy(u_hbm.at[row_v.at[jn]], bufs[wslot],
                                 sem_r[wslot])
        return c
    lax.fori_loop(0, (K_LBL + 2 + NB - 1) // NB, quad, 0)


_decoder_sc = pl.kernel(
    _decoder_body,
    out_type=jax.ShapeDtypeStruct((VLBL, D), jnp.float32),
    mesh=plsc.VectorSubcoreMesh(core_axis_name="c", subcore_axis_name="s"),
    scratch_types=(
        pltpu.VMEM((K_LBL, CH), jnp.int32),
        pltpu.VMEM((K_LBL, CH), jnp.int32),
        tuple(pltpu.VMEM((CH, D), jnp.float32) for _ in range(NB)),
        tuple(pltpu.SemaphoreType.DMA for _ in range(NB)),
        tuple(pltpu.SemaphoreType.DMA for _ in range(NB)),
        tuple(pltpu.SemaphoreType.DMA for _ in range(NB)),
    ),
)


_BLK = 1024


def _layer1_tc_body(agg_ref, cnt_ref, x_ref, wl_ref, bl_ref, wr_ref, o_ref):
    inv = 1.0 / jnp.maximum(cnt_ref[...], 1.0)
    mean = agg_ref[...] * inv
    h = (jnp.dot(mean, wl_ref[0], preferred_element_type=jnp.float32)
         + bl_ref[0]
         + jnp.dot(x_ref[...], wr_ref[0], preferred_element_type=jnp.float32))
    o_ref[...] = jnp.maximum(h, 0.0)


def _layer2_tc_body(agg_ref, cnt_ref, x_ref, wl_ref, bl_ref, wr_ref,
                    w1_ref, b1_ref, o_ref):
    inv = 1.0 / jnp.maximum(cnt_ref[...], 1.0)
    mean = agg_ref[...] * inv
    z = (jnp.dot(mean, wl_ref[0], preferred_element_type=jnp.float32)
         + bl_ref[0]
         + jnp.dot(x_ref[...], wr_ref[0], preferred_element_type=jnp.float32))
    o_ref[...] = jnp.dot(z, w1_ref[0], preferred_element_type=jnp.float32) + b1_ref[0]


def _row_blk(b):
    return (b, 0)


def _w_blk(b):
    return (b // (NPAD // _BLK), 0, 0)


_N_ROWS = 2 * NPAD
_node_specs = [
    pl.BlockSpec((_BLK, D), _row_blk),
    pl.BlockSpec((_BLK, 1), _row_blk),
    pl.BlockSpec((_BLK, D), _row_blk),
    pl.BlockSpec((1, D, D), _w_blk),
    pl.BlockSpec((1, 1, D), _w_blk),
    pl.BlockSpec((1, D, D), _w_blk),
]

_layer1_tc = pl.pallas_call(
    _layer1_tc_body,
    grid=(_N_ROWS // _BLK,),
    in_specs=_node_specs,
    out_specs=pl.BlockSpec((_BLK, D), _row_blk),
    out_shape=jax.ShapeDtypeStruct((_N_ROWS, D), jnp.float32),
)

_layer2_tc = pl.pallas_call(
    _layer2_tc_body,
    grid=(_N_ROWS // _BLK,),
    in_specs=_node_specs + [
        pl.BlockSpec((1, D, D), _w_blk),
        pl.BlockSpec((1, 1, D), _w_blk),
    ],
    out_specs=pl.BlockSpec((_BLK, D), _row_blk),
    out_shape=jax.ShapeDtypeStruct((_N_ROWS, D), jnp.float32),
)


def _final_tc_body(v_ref, w2_ref, b2_ref, o_ref):
    o_ref[...] = (jnp.sum(jnp.maximum(v_ref[...], 0.0) * w2_ref[...],
                          axis=1, keepdims=True) + b2_ref[0, 0])


_FBLK = 2048
_final_tc = pl.pallas_call(
    _final_tc_body,
    grid=(VLBL // _FBLK,),
    in_specs=[
        pl.BlockSpec((_FBLK, D), _row_blk),
        pl.BlockSpec((1, D), lambda b: (0, 0)),
        pl.BlockSpec((1, 1), lambda b: (0, 0)),
    ],
    out_specs=pl.BlockSpec((_FBLK, 1), _row_blk),
    out_shape=jax.ShapeDtypeStruct((VLBL, 1), jnp.float32),
)


def _prep_edges(src, dst, src_off):
    pad = EPT - E_EDGE
    s = jnp.concatenate([src.astype(jnp.int32) + src_off,
                         jnp.full((pad,), src_off, jnp.int32)])
    d = jnp.concatenate([dst.astype(jnp.int32),
                         jnp.full((pad,), N_NODE, jnp.int32)])
    return s.reshape(NTILE, K_CONV, CH), d.reshape(NTILE, K_CONV, CH)


def kernel(x_drug, x_disease, edge_index_dd, edge_index_dr, edge_label_index,
           Wl1_dd, bl1_dd, Wr1_dd, Wl1_dr, bl1_dr, Wr1_dr,
           Wl2_dd, bl2_dd, Wr2_dd, Wl2_dr, bl2_dr, Wr2_dr,
           W1, b1, W2, b2):
    f32 = jnp.float32
    pad_n = NPAD - N_NODE
    x_cat = jnp.concatenate([
        jnp.pad(x_drug, ((0, pad_n), (0, 0))),
        jnp.pad(x_disease, ((0, pad_n), (0, 0))),
    ]).astype(f32)

    s_dd, d_dd = _prep_edges(edge_index_dd[0], edge_index_dd[1], 0)
    s_dr, d_dr = _prep_edges(edge_index_dr[0], edge_index_dr[1], NPAD)
    src_a = jnp.stack([s_dd, s_dr])
    dst_a = jnp.stack([d_dd, d_dr])

    agg1, cnt = _make_conv(True)(x_cat, src_a, dst_a)
    cnt2d = cnt.reshape(-1, 1)

    wl1 = jnp.stack([Wl1_dr, Wl1_dd])
    bl1 = jnp.stack([bl1_dr, bl1_dd]).reshape(2, 1, D)
    wr1 = jnp.stack([Wr1_dr, Wr1_dd])
    h_cat = _layer1_tc(agg1, cnt2d, x_cat, wl1, bl1, wr1)

    (agg2,) = _make_conv(False)(h_cat, src_a, dst_a)
    wl2 = jnp.stack([Wl2_dr, Wl2_dd])
    bl2 = jnp.stack([bl2_dr, bl2_dd]).reshape(2, 1, D)
    wr2 = jnp.stack([Wr2_dr, Wr2_dd])
    w1s = jnp.stack([W1[:D], W1[D:]])
    b1s = jnp.stack([b1, jnp.zeros((D,), f32)]).reshape(2, 1, D)
    u_cat = _layer2_tc(agg2, cnt2d, h_cat, wl2, bl2, wr2, w1s, b1s)

    lpad = VLBL - E_LBL
    row = jnp.concatenate([edge_label_index[0].astype(jnp.int32),
                           jnp.zeros((lpad,), jnp.int32)])
    col = jnp.concatenate([edge_label_index[1].astype(jnp.int32) + NPAD,
                           jnp.full((lpad,), NPAD, jnp.int32)])
    row_a = row.reshape(NSC, NTILE, K_LBL, CH)
    col_a = col.reshape(NSC, NTILE, K_LBL, CH)
    v = _decoder_sc(u_cat, row_a, col_a)

    out = _final_tc(v, W2.reshape(1, D), b2.reshape(1, 1))
    return out[:E_LBL, 0]
